# baseline (speedup 1.0000x reference)
import dataclasses
import functools

import jax
import jax.numpy as jnp
from jax import lax
from jax.experimental import pallas as pl
from jax.experimental.pallas import tpu as pltpu
from jax.experimental.pallas import tpu_sc as plsc

N = 262144
B = 1024
NZ = 100
DE = 16
H = 64

A = 1024
NB = N // A
NW = 32
CH = N // NW
WIN = 2048
NWIN = CH // WIN

_SC_MESH = dict(core_axis_name="c", subcore_axis_name="s")


def _sc_compiler_params():
    cp = pltpu.CompilerParams()
    if "needs_layout_passes" in pltpu.CompilerParams.__dataclass_fields__:
        cp = dataclasses.replace(cp, needs_layout_passes=False)
    return cp


@functools.lru_cache(maxsize=None)
def _build_transpose_r():
    @functools.partial(
        pl.kernel,
        mesh=plsc.VectorSubcoreMesh(**_SC_MESH),
        compiler_params=_sc_compiler_params(),
        out_type=jax.ShapeDtypeStruct((3 * N,), jnp.float32),
        scratch_types=[
            pltpu.VMEM((3 * CH,), jnp.float32),
            pltpu.VMEM((CH,), jnp.float32),
            pltpu.VMEM((CH,), jnp.float32),
            pltpu.VMEM((CH,), jnp.float32),
            pltpu.SemaphoreType.DMA,
        ],
    )
    def _transpose_r(r_hbm, rt_hbm, rbuf, row0, row1, row2, sem):
        wid = lax.axis_index("c") * 16 + lax.axis_index("s")
        pltpu.async_copy(r_hbm.at[pl.ds(wid * CH * 3, CH * 3)], rbuf,
                         sem).wait()
        t3 = lax.iota(jnp.int32, 16) * 3

        @pl.loop(0, CH, step=16)
        def _(c):
            i3 = c * 3 + t3
            row0[pl.ds(c, 16)] = plsc.load_gather(rbuf, [i3])
            row1[pl.ds(c, 16)] = plsc.load_gather(rbuf, [i3 + 1])
            row2[pl.ds(c, 16)] = plsc.load_gather(rbuf, [i3 + 2])

        pltpu.sync_copy(row0, rt_hbm.at[pl.ds(wid * CH, CH)])
        pltpu.sync_copy(row1, rt_hbm.at[pl.ds(N + wid * CH, CH)])
        pltpu.sync_copy(row2, rt_hbm.at[pl.ds(2 * N + wid * CH, CH)])

    return _transpose_r


def _tc_body(rT_ref, z_ref, embT_ref, W1eT_ref, W1rT_ref, U_ref, w2c_ref,
             b1c_ref, b2_ref, ea_ref, gT_ref):
    rT = rT_ref[...]
    z = z_ref[0]

    MT = lax.dot_general(W1eT_ref[...], embT_ref[...],
                         (((1,), (0,)), ((), ())),
                         preferred_element_type=jnp.float32)

    idx = jnp.broadcast_to(z, (H, A))
    acc = jnp.take_along_axis(MT, idx, axis=1)
    acc = acc + lax.dot_general(W1rT_ref[...], rT.astype(jnp.bfloat16),
                                (((1,), (0,)), ((), ())),
                                preferred_element_type=jnp.float32)
    hpre = acc + b1c_ref[...]
    h = jnp.tanh(hpre)

    w2c = w2c_ref[...]
    hw = h * w2c
    d = w2c - h * hw

    X = jnp.concatenate([h, d], axis=0).astype(jnp.bfloat16)
    Y = lax.dot_general(U_ref[...], X, (((1,), (0,)), ((), ())),
                        preferred_element_type=jnp.float32)

    ea_ref[...] = (Y[0:1] + b2_ref[...]).reshape(1, 1, A)
    gT_ref[...] = Y[1:4]


def _tc_stage(rT, z3, embT, W1eT, W1rT, U, w2c, b1c, b2r):
    return pl.pallas_call(
        _tc_body,
        grid=(NB,),
        in_specs=[
            pl.BlockSpec((3, A), lambda i: (0, i)),
            pl.BlockSpec((1, 1, A), lambda i: (i, 0, 0)),
            pl.BlockSpec((16, 128), lambda i: (0, 0)),
            pl.BlockSpec((64, 16), lambda i: (0, 0)),
            pl.BlockSpec((64, 3), lambda i: (0, 0)),
            pl.BlockSpec((8, 128), lambda i: (0, 0)),
            pl.BlockSpec((64, 1), lambda i: (0, 0)),
            pl.BlockSpec((64, 1), lambda i: (0, 0)),
            pl.BlockSpec((1, 1), lambda i: (0, 0)),
        ],
        out_specs=[
            pl.BlockSpec((1, 1, A), lambda i: (i, 0, 0)),
            pl.BlockSpec((3, A), lambda i: (0, i)),
        ],
        out_shape=[
            jax.ShapeDtypeStruct((NB, 1, A), jnp.float32),
            jax.ShapeDtypeStruct((3, N), jnp.float32),
        ],
    )(rT, z3, embT, W1eT, W1rT, U, w2c, b1c, b2r)


@functools.lru_cache(maxsize=None)
def _build_seg_sum():
    @functools.partial(
        pl.kernel,
        mesh=plsc.VectorSubcoreMesh(**_SC_MESH),
        compiler_params=_sc_compiler_params(),
        out_type=(
            jax.ShapeDtypeStruct((NW * B,), jnp.float32),
            jax.ShapeDtypeStruct((NW * 48,), jnp.float32),
        ),
        scratch_types=[
            pltpu.VMEM((CH,), jnp.float32),
            pltpu.VMEM((CH,), jnp.int32),
            pltpu.VMEM((3 * CH,), jnp.float32),
            pltpu.VMEM((CH,), jnp.float32),
            pltpu.VMEM((B,), jnp.float32),
            pltpu.VMEM((48,), jnp.float32),
            pltpu.SemaphoreType.DMA,
        ],
    )
    def _seg_sum(ea_hbm, batch_hbm, v_hbm, m_hbm, out_hbm, pp_hbm,
                 vals_v, ids_v, v_v, m_v, acc_v, pst_v, sem):
        wid = lax.axis_index("c") * 16 + lax.axis_index("s")
        base = wid * CH
        cp1 = pltpu.async_copy(ea_hbm.at[pl.ds(base, CH)], vals_v, sem)
        cp2 = pltpu.async_copy(batch_hbm.at[pl.ds(base, CH)], ids_v, sem)
        cp3 = pltpu.async_copy(v_hbm.at[pl.ds(base * 3, CH * 3)], v_v, sem)
        cp4 = pltpu.async_copy(m_hbm.at[pl.ds(base, CH)], m_v, sem)

        zero16 = jnp.zeros((16,), jnp.float32)

        @pl.loop(0, B, step=16)
        def _(k):
            acc_v[pl.ds(k, 16)] = zero16

        cp1.wait()
        cp2.wait()
        cp3.wait()
        cp4.wait()

        iota = lax.iota(jnp.int32, 16)
        t3 = iota * 3
        m_last = iota == 15
        m_first = iota == 0
        sh_next = jnp.minimum(iota + 1, 15)
        sh_prev = jnp.maximum(iota - 1, 0)

        pst_v[pl.ds(0, 16)] = zero16
        pst_v[pl.ds(16, 16)] = zero16
        pst_v[pl.ds(32, 16)] = zero16

        @pl.loop(0, CH, step=16)
        def _(c):
            ids = ids_v[pl.ds(c, 16)]
            i3 = c * 3 + t3
            va = plsc.load_gather(v_v, [i3])
            vb = plsc.load_gather(v_v, [i3 + 1])
            vc = plsc.load_gather(v_v, [i3 + 2])
            mm = m_v[pl.ds(c, 16)]
            vals = vals_v[pl.ds(c, 16)] + 0.5 * mm * (va * va + vb * vb
                                                      + vc * vc)
            ids_n = plsc.load_gather(ids_v, [c + sh_next])
            ids_p = plsc.load_gather(ids_v, [c + sh_prev])
            bnd = (ids != ids_n) | m_last
            stt = (ids != ids_p) | m_first
            ps = plsc.cumsum(vals)
            plsc.addupdate_scatter(acc_v, [ids], ps, mask=bnd)
            plsc.addupdate_scatter(acc_v, [ids], vals - ps, mask=stt)
            pst_v[pl.ds(0, 16)] = pst_v[pl.ds(0, 16)] + mm * va
            pst_v[pl.ds(16, 16)] = pst_v[pl.ds(16, 16)] + mm * vb
            pst_v[pl.ds(32, 16)] = pst_v[pl.ds(32, 16)] + mm * vc

        pltpu.sync_copy(acc_v, out_hbm.at[pl.ds(wid * B, B)])
        pltpu.sync_copy(pst_v, pp_hbm.at[pl.ds(wid * 48, 48)])

    return _seg_sum


@functools.lru_cache(maxsize=None)
def _build_j_assemble():
    @functools.partial(
        pl.kernel,
        mesh=plsc.VectorSubcoreMesh(**_SC_MESH),
        compiler_params=_sc_compiler_params(),
        out_type=(
            jax.ShapeDtypeStruct((B,), jnp.float32),
            jax.ShapeDtypeStruct((48,), jnp.float32),
            jax.ShapeDtypeStruct((N * 6,), jnp.float32),
        ),
        scratch_types=[
            pltpu.VMEM((NW * B,), jnp.float32),
            pltpu.VMEM((NW * 48,), jnp.float32),
            pltpu.VMEM((B,), jnp.float32),
            pltpu.VMEM((B,), jnp.float32),
            pltpu.VMEM((48,), jnp.float32),
            pltpu.VMEM((WIN,), jnp.int32),
            pltpu.VMEM((WIN,), jnp.float32),
            pltpu.VMEM((WIN,), jnp.float32),
            pltpu.VMEM((WIN,), jnp.float32),
            pltpu.VMEM((WIN,), jnp.float32),
            pltpu.VMEM((WIN * 6,), jnp.float32),
            pltpu.SemaphoreType.DMA,
        ],
    )
    def _j_assemble(parts_hbm, pp_hbm, e0_hbm, batch_hbm, m_hbm, gT_hbm,
                    ec_hbm, pout_hbm, j_hbm,
                    parts_v, pp_v, e0_v, ec_v, pst_v, ids_v, m_v,
                    gx_v, gy_v, gz_v, jst_v, sem):
        wid = lax.axis_index("c") * 16 + lax.axis_index("s")

        pltpu.async_copy(parts_hbm, parts_v, sem).wait()
        pltpu.async_copy(pp_hbm, pp_v, sem).wait()
        pltpu.async_copy(e0_hbm, e0_v, sem).wait()

        zero16 = jnp.zeros((16,), jnp.float32)
        px = zero16
        py = zero16
        pz = zero16
        for w in range(NW):
            px = px + pp_v[pl.ds(w * 48, 16)]
            py = py + pp_v[pl.ds(w * 48 + 16, 16)]
            pz = pz + pp_v[pl.ds(w * 48 + 32, 16)]
        px = zero16 + jnp.sum(px)
        py = zero16 + jnp.sum(py)
        pz = zero16 + jnp.sum(pz)

        @pl.loop(0, B, step=16)
        def _(k):
            a16 = -e0_v[pl.ds(k, 16)]
            for r in range(NW):
                a16 = a16 + parts_v[pl.ds(k + r * B, 16)]
            ec_v[pl.ds(k, 16)] = a16

        pltpu.sync_copy(ec_v.at[pl.ds(wid * 32, 32)],
                        ec_hbm.at[pl.ds(wid * 32, 32)])

        @pl.when(wid == 0)
        def _():
            pst_v[pl.ds(0, 16)] = px
            pst_v[pl.ds(16, 16)] = py
            pst_v[pl.ds(32, 16)] = pz
            pltpu.sync_copy(pst_v, pout_hbm)

        iota = lax.iota(jnp.int32, 16)

        for t in range(NWIN):
            gbase = wid * CH + t * WIN
            pltpu.sync_copy(batch_hbm.at[pl.ds(gbase, WIN)], ids_v)
            pltpu.sync_copy(m_hbm.at[pl.ds(gbase, WIN)], m_v)
            pltpu.sync_copy(gT_hbm.at[pl.ds(gbase, WIN)], gx_v)
            pltpu.sync_copy(gT_hbm.at[pl.ds(N + gbase, WIN)], gy_v)
            pltpu.sync_copy(gT_hbm.at[pl.ds(2 * N + gbase, WIN)], gz_v)

            @pl.loop(0, WIN, step=16)
            def _(c):
                ids = ids_v[pl.ds(c, 16)]
                eb = plsc.load_gather(ec_v, [ids])
                mm = m_v[pl.ds(c, 16)]
                a = eb + mm
                idx6 = (c + iota) * 6
                gx = gx_v[pl.ds(c, 16)]
                gy = gy_v[pl.ds(c, 16)]
                gz = gz_v[pl.ds(c, 16)]
                plsc.store_scatter(jst_v, [idx6], gx * eb)
                plsc.store_scatter(jst_v, [idx6 + 1], gy * eb)
                plsc.store_scatter(jst_v, [idx6 + 2], gz * eb)
                plsc.store_scatter(jst_v, [idx6 + 3], a * px)
                plsc.store_scatter(jst_v, [idx6 + 4], a * py)
                plsc.store_scatter(jst_v, [idx6 + 5], a * pz)

            pltpu.sync_copy(jst_v, j_hbm.at[pl.ds(gbase * 6, WIN * 6)])

    return _j_assemble


def kernel(r, v, batch, z, m, emb, W1, b1, W2, b2, E0):
    f32 = jnp.float32
    bf16 = jnp.bfloat16
    r_flat = r.astype(f32).reshape(3 * N)
    v_flat = v.astype(f32).reshape(3 * N)
    batch_i = batch.astype(jnp.int32)
    z3 = z.astype(jnp.int32).reshape(NB, 1, A)
    m_f = m.astype(f32)

    embT = jnp.pad(emb.astype(f32).T, ((0, 0), (0, 128 - NZ)))
    W1eT = W1[3:].astype(f32).T
    W1rT = W1[:3].astype(f32).T.astype(bf16)
    w2c = W2.astype(f32)
    U = jnp.zeros((8, 128), f32)
    U = U.at[0, :64].set(W2.astype(f32)[:, 0])
    U = U.at[1:4, 64:].set(W1[:3].astype(f32))
    U = U.astype(bf16)
    b1c = b1.astype(f32)[:, None]
    b2r = b2.astype(f32).reshape(1, 1)

    rTf = _build_transpose_r()(r_flat)
    ea3, gT = _tc_stage(rTf.reshape(3, N), z3, embT, W1eT, W1rT, U,
                        w2c, b1c, b2r)
    ea_flat = ea3.reshape(N)

    parts, pparts = _build_seg_sum()(ea_flat, batch_i, v_flat, m_f)
    Ec, p48, Jf = _build_j_assemble()(parts, pparts, E0.astype(f32),
                                      batch_i, m_f, gT.reshape(3 * N))
    p_vec = jnp.stack([p48[0], p48[16], p48[32]])
    c = jnp.concatenate([Ec, p_vec])
    return (c, Jf.reshape(N, 6))


# speedup vs baseline: 3.6375x; 3.6375x over previous
import dataclasses
import functools

import jax
import jax.numpy as jnp
from jax import lax
from jax.experimental import pallas as pl
from jax.experimental.pallas import tpu as pltpu
from jax.experimental.pallas import tpu_sc as plsc

N = 262144
B = 1024
NZ = 100
DE = 16
H = 64

A = 2048
NB = N // A
A2 = 8192
NB2 = N // A2
NW = 32
CH = N // NW

_SC_MESH = dict(core_axis_name="c", subcore_axis_name="s")


def _sc_compiler_params():
    cp = pltpu.CompilerParams()
    if "needs_layout_passes" in pltpu.CompilerParams.__dataclass_fields__:
        cp = dataclasses.replace(cp, needs_layout_passes=False)
    return cp


def _tc_body(rx_ref, ry_ref, rz_ref, z_ref, embT_ref, W1eT_ref, W1rT_ref,
             U_ref, w2c_ref, b1c_ref, b2_ref,
             ea_ref, gx_ref, gy_ref, gz_ref):
    rT = jnp.concatenate([rx_ref[0], ry_ref[0], rz_ref[0]], axis=0)
    z = z_ref[0]

    MT = lax.dot_general(W1eT_ref[...], embT_ref[...],
                         (((1,), (0,)), ((), ())),
                         preferred_element_type=jnp.float32)

    idx = jnp.broadcast_to(z, (H, A))
    acc = jnp.take_along_axis(MT, idx, axis=1)
    acc = acc + lax.dot_general(W1rT_ref[...], rT.astype(jnp.bfloat16),
                                (((1,), (0,)), ((), ())),
                                preferred_element_type=jnp.float32)
    hpre = acc + b1c_ref[...]
    h = jnp.tanh(hpre)

    w2c = w2c_ref[...]
    hw = h * w2c
    d = w2c - h * hw

    X = jnp.concatenate([h, d], axis=0).astype(jnp.bfloat16)
    Y = lax.dot_general(U_ref[...], X, (((1,), (0,)), ((), ())),
                        preferred_element_type=jnp.float32)

    ea_ref[...] = (Y[0:1] + b2_ref[...]).reshape(1, 1, A)
    gx_ref[...] = Y[1:2].reshape(1, 1, A)
    gy_ref[...] = Y[2:3].reshape(1, 1, A)
    gz_ref[...] = Y[3:4].reshape(1, 1, A)


def _tc_stage(rx3, ry3, rz3, z3, embT, W1eT, W1rT, U, w2c, b1c, b2r):
    row = pl.BlockSpec((1, 1, A), lambda i: (i, 0, 0))
    full = lambda s: pl.BlockSpec(s, lambda i: tuple(0 for _ in s))
    out1 = jax.ShapeDtypeStruct((NB, 1, A), jnp.float32)
    return pl.pallas_call(
        _tc_body,
        grid=(NB,),
        in_specs=[row, row, row, row,
                  full((16, 128)), full((64, 16)), full((64, 3)),
                  full((8, 128)), full((64, 1)), full((64, 1)),
                  full((1, 1))],
        out_specs=[row, row, row, row],
        out_shape=[out1, out1, out1, out1],
    )(rx3, ry3, rz3, z3, embT, W1eT, W1rT, U, w2c, b1c, b2r)


def _tc2_body(eb_ref, m_ref, gx_ref, gy_ref, gz_ref, p_ref, jt_ref):
    eb = eb_ref[0]
    m2 = m_ref[0]
    a = eb + m2
    px = jnp.broadcast_to(p_ref[0, :, 0:1], (1, A2))
    py = jnp.broadcast_to(p_ref[0, :, 16:17], (1, A2))
    pz = jnp.broadcast_to(p_ref[0, :, 32:33], (1, A2))
    jt_ref[...] = jnp.concatenate(
        [gx_ref[0] * eb, gy_ref[0] * eb, gz_ref[0] * eb,
         a * px, a * py, a * pz], axis=0)


def _tc2_stage(eb3, m3, gx3, gy3, gz3, p48):
    row = pl.BlockSpec((1, 1, A2), lambda i: (i, 0, 0))
    return pl.pallas_call(
        _tc2_body,
        grid=(NB2,),
        in_specs=[row, row, row, row, row,
                  pl.BlockSpec((1, 1, 48), lambda i: (0, 0, 0))],
        out_specs=pl.BlockSpec((6, A2), lambda i: (0, i)),
        out_shape=jax.ShapeDtypeStruct((6, N), jnp.float32),
    )(eb3, m3, gx3, gy3, gz3, p48)


@functools.lru_cache(maxsize=None)
def _build_kin_seg_sum():
    @functools.partial(
        pl.kernel,
        mesh=plsc.VectorSubcoreMesh(**_SC_MESH),
        compiler_params=_sc_compiler_params(),
        out_type=(
            jax.ShapeDtypeStruct((NW * B,), jnp.float32),
            jax.ShapeDtypeStruct((NW * 48,), jnp.float32),
        ),
        scratch_types=[
            pltpu.VMEM((CH,), jnp.int32),
            pltpu.VMEM((CH,), jnp.float32),
            pltpu.VMEM((CH,), jnp.float32),
            pltpu.VMEM((CH,), jnp.float32),
            pltpu.VMEM((CH,), jnp.float32),
            pltpu.VMEM((B,), jnp.float32),
            pltpu.VMEM((48,), jnp.float32),
            pltpu.SemaphoreType.DMA,
        ],
    )
    def _kin_seg(batch_hbm, vx_hbm, vy_hbm, vz_hbm, m_hbm, out_hbm, pp_hbm,
                 ids_v, vx_v, vy_v, vz_v, m_v, acc_v, pst_v, sem):
        wid = lax.axis_index("c") * 16 + lax.axis_index("s")
        base = wid * CH
        cps = [pltpu.async_copy(batch_hbm.at[pl.ds(base, CH)], ids_v, sem),
               pltpu.async_copy(vx_hbm.at[pl.ds(base, CH)], vx_v, sem),
               pltpu.async_copy(vy_hbm.at[pl.ds(base, CH)], vy_v, sem),
               pltpu.async_copy(vz_hbm.at[pl.ds(base, CH)], vz_v, sem),
               pltpu.async_copy(m_hbm.at[pl.ds(base, CH)], m_v, sem)]

        zero16 = jnp.zeros((16,), jnp.float32)

        @pl.loop(0, B, step=16)
        def _(k):
            acc_v[pl.ds(k, 16)] = zero16

        pst_v[pl.ds(0, 16)] = zero16
        pst_v[pl.ds(16, 16)] = zero16
        pst_v[pl.ds(32, 16)] = zero16

        for cp in cps:
            cp.wait()

        iota = lax.iota(jnp.int32, 16)
        m_last = iota == 15
        m_first = iota == 0
        sh_next = jnp.minimum(iota + 1, 15)
        sh_prev = jnp.maximum(iota - 1, 0)

        @pl.loop(0, CH, step=16)
        def _(c):
            ids = ids_v[pl.ds(c, 16)]
            va = vx_v[pl.ds(c, 16)]
            vb = vy_v[pl.ds(c, 16)]
            vc = vz_v[pl.ds(c, 16)]
            mm = m_v[pl.ds(c, 16)]
            vals = 0.5 * mm * (va * va + vb * vb + vc * vc)
            ids_n = plsc.load_gather(ids_v, [c + sh_next])
            ids_p = plsc.load_gather(ids_v, [c + sh_prev])
            bnd = (ids != ids_n) | m_last
            stt = (ids != ids_p) | m_first
            ps = plsc.cumsum(vals)
            plsc.addupdate_scatter(acc_v, [ids], ps, mask=bnd)
            plsc.addupdate_scatter(acc_v, [ids], vals - ps, mask=stt)
            pst_v[pl.ds(0, 16)] = pst_v[pl.ds(0, 16)] + mm * va
            pst_v[pl.ds(16, 16)] = pst_v[pl.ds(16, 16)] + mm * vb
            pst_v[pl.ds(32, 16)] = pst_v[pl.ds(32, 16)] + mm * vc

        pltpu.sync_copy(acc_v, out_hbm.at[pl.ds(wid * B, B)])
        pltpu.sync_copy(pst_v, pp_hbm.at[pl.ds(wid * 48, 48)])

    return _kin_seg


@functools.lru_cache(maxsize=None)
def _build_seg_sum():
    @functools.partial(
        pl.kernel,
        mesh=plsc.VectorSubcoreMesh(**_SC_MESH),
        compiler_params=_sc_compiler_params(),
        out_type=jax.ShapeDtypeStruct((NW * B,), jnp.float32),
        scratch_types=[
            pltpu.VMEM((CH,), jnp.float32),
            pltpu.VMEM((CH,), jnp.int32),
            pltpu.VMEM((B,), jnp.float32),
            pltpu.SemaphoreType.DMA,
        ],
    )
    def _seg_sum(ea_hbm, batch_hbm, out_hbm, vals_v, ids_v, acc_v, sem):
        wid = lax.axis_index("c") * 16 + lax.axis_index("s")
        base = wid * CH
        cp1 = pltpu.async_copy(ea_hbm.at[pl.ds(base, CH)], vals_v, sem)
        cp2 = pltpu.async_copy(batch_hbm.at[pl.ds(base, CH)], ids_v, sem)

        zero16 = jnp.zeros((16,), jnp.float32)

        @pl.loop(0, B, step=16)
        def _(k):
            acc_v[pl.ds(k, 16)] = zero16

        cp1.wait()
        cp2.wait()

        iota = lax.iota(jnp.int32, 16)
        m_last = iota == 15
        m_first = iota == 0
        sh_next = jnp.minimum(iota + 1, 15)
        sh_prev = jnp.maximum(iota - 1, 0)

        @pl.loop(0, CH, step=16)
        def _(c):
            ids = ids_v[pl.ds(c, 16)]
            vals = vals_v[pl.ds(c, 16)]
            ids_n = plsc.load_gather(ids_v, [c + sh_next])
            ids_p = plsc.load_gather(ids_v, [c + sh_prev])
            bnd = (ids != ids_n) | m_last
            stt = (ids != ids_p) | m_first
            ps = plsc.cumsum(vals)
            plsc.addupdate_scatter(acc_v, [ids], ps, mask=bnd)
            plsc.addupdate_scatter(acc_v, [ids], vals - ps, mask=stt)

        pltpu.sync_copy(acc_v, out_hbm.at[pl.ds(wid * B, B)])

    return _seg_sum


@functools.lru_cache(maxsize=None)
def _build_ec_gather():
    @functools.partial(
        pl.kernel,
        mesh=plsc.VectorSubcoreMesh(**_SC_MESH),
        compiler_params=_sc_compiler_params(),
        out_type=(
            jax.ShapeDtypeStruct((B,), jnp.float32),
            jax.ShapeDtypeStruct((48,), jnp.float32),
            jax.ShapeDtypeStruct((N,), jnp.float32),
        ),
        scratch_types=[
            pltpu.VMEM((NW * B,), jnp.float32),
            pltpu.VMEM((NW * B,), jnp.float32),
            pltpu.VMEM((NW * 48,), jnp.float32),
            pltpu.VMEM((B,), jnp.float32),
            pltpu.VMEM((B,), jnp.float32),
            pltpu.VMEM((48,), jnp.float32),
            pltpu.VMEM((CH,), jnp.int32),
            pltpu.VMEM((CH,), jnp.float32),
            pltpu.SemaphoreType.DMA,
        ],
    )
    def _ec_gather(pk_hbm, pp_hbm, ppart_hbm, e0_hbm, batch_hbm,
                   ec_hbm, pout_hbm, eb_hbm,
                   pk_v, pp_v, ppart_v, e0_v, ec_v, pst_v, ids_v, eb_v, sem):
        wid = lax.axis_index("c") * 16 + lax.axis_index("s")
        base = wid * CH

        cp0 = pltpu.async_copy(batch_hbm.at[pl.ds(base, CH)], ids_v, sem)
        pltpu.async_copy(pk_hbm, pk_v, sem).wait()
        pltpu.async_copy(pp_hbm, pp_v, sem).wait()
        pltpu.async_copy(ppart_hbm, ppart_v, sem).wait()
        pltpu.async_copy(e0_hbm, e0_v, sem).wait()

        zero16 = jnp.zeros((16,), jnp.float32)
        px = zero16
        py = zero16
        pz = zero16
        for w in range(NW):
            px = px + ppart_v[pl.ds(w * 48, 16)]
            py = py + ppart_v[pl.ds(w * 48 + 16, 16)]
            pz = pz + ppart_v[pl.ds(w * 48 + 32, 16)]
        px = zero16 + jnp.sum(px)
        py = zero16 + jnp.sum(py)
        pz = zero16 + jnp.sum(pz)

        @pl.loop(0, B, step=16)
        def _(k):
            a16 = -e0_v[pl.ds(k, 16)]
            for r in range(NW):
                a16 = a16 + pk_v[pl.ds(k + r * B, 16)]
                a16 = a16 + pp_v[pl.ds(k + r * B, 16)]
            ec_v[pl.ds(k, 16)] = a16

        pltpu.sync_copy(ec_v.at[pl.ds(wid * 32, 32)],
                        ec_hbm.at[pl.ds(wid * 32, 32)])

        @pl.when(wid == 0)
        def _():
            pst_v[pl.ds(0, 16)] = px
            pst_v[pl.ds(16, 16)] = py
            pst_v[pl.ds(32, 16)] = pz
            pltpu.sync_copy(pst_v, pout_hbm)

        cp0.wait()

        @pl.loop(0, CH, step=16)
        def _(c):
            ids = ids_v[pl.ds(c, 16)]
            eb_v[pl.ds(c, 16)] = plsc.load_gather(ec_v, [ids])

        pltpu.sync_copy(eb_v, eb_hbm.at[pl.ds(base, CH)])

    return _ec_gather


def kernel(r, v, batch, z, m, emb, W1, b1, W2, b2, E0):
    f32 = jnp.float32
    bf16 = jnp.bfloat16
    r = r.astype(f32)
    v = v.astype(f32)
    rx3 = r[:, 0].reshape(NB, 1, A)
    ry3 = r[:, 1].reshape(NB, 1, A)
    rz3 = r[:, 2].reshape(NB, 1, A)
    vx = v[:, 0]
    vy = v[:, 1]
    vz = v[:, 2]
    batch_i = batch.astype(jnp.int32)
    z3 = z.astype(jnp.int32).reshape(NB, 1, A)
    m_f = m.astype(f32)

    embT = jnp.pad(emb.astype(f32).T, ((0, 0), (0, 128 - NZ)))
    W1eT = W1[3:].astype(f32).T
    W1rT = W1[:3].astype(f32).T.astype(bf16)
    w2c = W2.astype(f32)
    U = jnp.zeros((8, 128), f32)
    U = U.at[0, :64].set(W2.astype(f32)[:, 0])
    U = U.at[1:4, 64:].set(W1[:3].astype(f32))
    U = U.astype(bf16)
    b1c = b1.astype(f32)[:, None]
    b2r = b2.astype(f32).reshape(1, 1)

    parts_kin, pparts = _build_kin_seg_sum()(batch_i, vx, vy, vz, m_f)

    ea3, gx3, gy3, gz3 = _tc_stage(rx3, ry3, rz3, z3, embT, W1eT, W1rT, U,
                                   w2c, b1c, b2r)
    parts_pot = _build_seg_sum()(ea3.reshape(N), batch_i)

    Ec, p48, Eb = _build_ec_gather()(parts_kin, parts_pot, pparts,
                                     E0.astype(f32), batch_i)

    JT = _tc2_stage(Eb.reshape(NB2, 1, A2), m_f.reshape(NB2, 1, A2),
                    gx3.reshape(NB2, 1, A2), gy3.reshape(NB2, 1, A2),
                    gz3.reshape(NB2, 1, A2), p48.reshape(1, 1, 48))

    p_vec = jnp.stack([p48[0], p48[16], p48[32]])
    c = jnp.concatenate([Ec, p_vec])
    return (c, JT.T)


# speedup vs baseline: 4.6979x; 1.2915x over previous
import dataclasses
import functools

import jax
import jax.numpy as jnp
from jax import lax
from jax.experimental import pallas as pl
from jax.experimental.pallas import tpu as pltpu
from jax.experimental.pallas import tpu_sc as plsc

N = 262144
B = 1024
NZ = 100
DE = 16
H = 64

A = 16384
NB = N // A
A2 = 8192
NB2 = N // A2
NW = 32
CH = N // NW

_SC_MESH = dict(core_axis_name="c", subcore_axis_name="s")


def _sc_compiler_params():
    cp = pltpu.CompilerParams()
    if "needs_layout_passes" in pltpu.CompilerParams.__dataclass_fields__:
        cp = dataclasses.replace(cp, needs_layout_passes=False)
    return cp


AH = A // 2


def _tc_body(rx_ref, ry_ref, rz_ref, z_ref, embT_ref, W1eT_ref, W1rT_ref,
             U_ref, w2c_ref, b1c_ref, b2_ref,
             ea_ref, gx_ref, gy_ref, gz_ref, MT_ref):
    i = pl.program_id(0)

    @pl.when(i == 0)
    def _():
        MT_ref[...] = lax.dot_general(W1eT_ref[...], embT_ref[...],
                                      (((1,), (0,)), ((), ())),
                                      preferred_element_type=jnp.float32)

    MT = MT_ref[...]
    w2c = w2c_ref[...]
    b1c = b1c_ref[...]

    for s in range(2):
        sl = slice(s * AH, (s + 1) * AH)
        rT = jnp.concatenate([rx_ref[0][:, sl], ry_ref[0][:, sl],
                              rz_ref[0][:, sl]], axis=0)
        z = z_ref[0][:, sl]

        idx = jnp.broadcast_to(z, (H, AH))
        acc = jnp.take_along_axis(MT, idx, axis=1)
        acc = acc + lax.dot_general(W1rT_ref[...], rT,
                                    (((1,), (0,)), ((), ())),
                                    preferred_element_type=jnp.float32)
        hpre = acc + b1c
        h = jnp.tanh(hpre)

        hw = h * w2c
        d = w2c - h * hw

        X = jnp.concatenate([h, d], axis=0)
        Y = lax.dot_general(U_ref[...], X, (((1,), (0,)), ((), ())),
                            preferred_element_type=jnp.float32)

        ea_ref[0, :, sl] = Y[0:1] + b2_ref[...]
        gx_ref[0, :, sl] = Y[1:2]
        gy_ref[0, :, sl] = Y[2:3]
        gz_ref[0, :, sl] = Y[3:4]


def _tc_stage(rx3, ry3, rz3, z3, embT, W1eT, W1rT, U, w2c, b1c, b2r):
    row = pl.BlockSpec((1, 1, A), lambda i: (i, 0, 0))
    full = lambda s: pl.BlockSpec(s, lambda i: tuple(0 for _ in s))
    out1 = jax.ShapeDtypeStruct((NB, 1, A), jnp.float32)
    return pl.pallas_call(
        _tc_body,
        grid=(NB,),
        in_specs=[row, row, row, row,
                  full((16, 128)), full((64, 16)), full((64, 3)),
                  full((8, 128)), full((64, 1)), full((64, 1)),
                  full((1, 1))],
        out_specs=[row, row, row, row],
        out_shape=[out1, out1, out1, out1],
        scratch_shapes=[pltpu.VMEM((64, 128), jnp.float32)],
    )(rx3, ry3, rz3, z3, embT, W1eT, W1rT, U, w2c, b1c, b2r)


def _tc2_body(eb_ref, m_ref, gx_ref, gy_ref, gz_ref, p_ref, jt_ref):
    eb = eb_ref[0]
    m2 = m_ref[0]
    a = eb + m2
    px = jnp.broadcast_to(p_ref[0, :, 0:1], (1, A2))
    py = jnp.broadcast_to(p_ref[0, :, 16:17], (1, A2))
    pz = jnp.broadcast_to(p_ref[0, :, 32:33], (1, A2))
    jt_ref[...] = jnp.concatenate(
        [gx_ref[0] * eb, gy_ref[0] * eb, gz_ref[0] * eb,
         a * px, a * py, a * pz], axis=0)


def _tc2_stage(eb3, m3, gx3, gy3, gz3, p48):
    row = pl.BlockSpec((1, 1, A2), lambda i: (i, 0, 0))
    return pl.pallas_call(
        _tc2_body,
        grid=(NB2,),
        in_specs=[row, row, row, row, row,
                  pl.BlockSpec((1, 1, 48), lambda i: (0, 0, 0))],
        out_specs=pl.BlockSpec((6, A2), lambda i: (0, i)),
        out_shape=jax.ShapeDtypeStruct((6, N), jnp.float32),
    )(eb3, m3, gx3, gy3, gz3, p48)


@functools.lru_cache(maxsize=None)
def _build_kin_seg_sum():
    @functools.partial(
        pl.kernel,
        mesh=plsc.VectorSubcoreMesh(**_SC_MESH),
        compiler_params=_sc_compiler_params(),
        out_type=(
            jax.ShapeDtypeStruct((NW * B,), jnp.float32),
            jax.ShapeDtypeStruct((NW * 48,), jnp.float32),
        ),
        scratch_types=[
            pltpu.VMEM((CH,), jnp.int32),
            pltpu.VMEM((CH,), jnp.float32),
            pltpu.VMEM((CH,), jnp.float32),
            pltpu.VMEM((CH,), jnp.float32),
            pltpu.VMEM((CH,), jnp.float32),
            pltpu.VMEM((B,), jnp.float32),
            pltpu.VMEM((48,), jnp.float32),
            pltpu.SemaphoreType.DMA,
        ],
    )
    def _kin_seg(batch_hbm, vx_hbm, vy_hbm, vz_hbm, m_hbm, out_hbm, pp_hbm,
                 ids_v, vx_v, vy_v, vz_v, m_v, acc_v, pst_v, sem):
        wid = lax.axis_index("c") * 16 + lax.axis_index("s")
        base = wid * CH
        cps = [pltpu.async_copy(batch_hbm.at[pl.ds(base, CH)], ids_v, sem),
               pltpu.async_copy(vx_hbm.at[pl.ds(base, CH)], vx_v, sem),
               pltpu.async_copy(vy_hbm.at[pl.ds(base, CH)], vy_v, sem),
               pltpu.async_copy(vz_hbm.at[pl.ds(base, CH)], vz_v, sem),
               pltpu.async_copy(m_hbm.at[pl.ds(base, CH)], m_v, sem)]

        zero16 = jnp.zeros((16,), jnp.float32)

        @pl.loop(0, B, step=16)
        def _(k):
            acc_v[pl.ds(k, 16)] = zero16

        pst_v[pl.ds(0, 16)] = zero16
        pst_v[pl.ds(16, 16)] = zero16
        pst_v[pl.ds(32, 16)] = zero16

        for cp in cps:
            cp.wait()

        iota = lax.iota(jnp.int32, 16)
        m_last = iota == 15
        m_first = iota == 0
        sh_next = jnp.minimum(iota + 1, 15)
        sh_prev = jnp.maximum(iota - 1, 0)

        @pl.loop(0, CH, step=16)
        def _(c):
            ids = ids_v[pl.ds(c, 16)]
            va = vx_v[pl.ds(c, 16)]
            vb = vy_v[pl.ds(c, 16)]
            vc = vz_v[pl.ds(c, 16)]
            mm = m_v[pl.ds(c, 16)]
            vals = 0.5 * mm * (va * va + vb * vb + vc * vc)
            ids_n = plsc.load_gather(ids_v, [c + sh_next])
            ids_p = plsc.load_gather(ids_v, [c + sh_prev])
            bnd = (ids != ids_n) | m_last
            stt = (ids != ids_p) | m_first
            ps = plsc.cumsum(vals)
            plsc.addupdate_scatter(acc_v, [ids], ps, mask=bnd)
            plsc.addupdate_scatter(acc_v, [ids], vals - ps, mask=stt)
            pst_v[pl.ds(0, 16)] = pst_v[pl.ds(0, 16)] + mm * va
            pst_v[pl.ds(16, 16)] = pst_v[pl.ds(16, 16)] + mm * vb
            pst_v[pl.ds(32, 16)] = pst_v[pl.ds(32, 16)] + mm * vc

        pltpu.sync_copy(acc_v, out_hbm.at[pl.ds(wid * B, B)])
        pltpu.sync_copy(pst_v, pp_hbm.at[pl.ds(wid * 48, 48)])

    return _kin_seg


@functools.lru_cache(maxsize=None)
def _build_seg_sum():
    @functools.partial(
        pl.kernel,
        mesh=plsc.VectorSubcoreMesh(**_SC_MESH),
        compiler_params=_sc_compiler_params(),
        out_type=jax.ShapeDtypeStruct((NW * B,), jnp.float32),
        scratch_types=[
            pltpu.VMEM((CH,), jnp.float32),
            pltpu.VMEM((CH,), jnp.int32),
            pltpu.VMEM((B,), jnp.float32),
            pltpu.SemaphoreType.DMA,
        ],
    )
    def _seg_sum(ea_hbm, batch_hbm, out_hbm, vals_v, ids_v, acc_v, sem):
        wid = lax.axis_index("c") * 16 + lax.axis_index("s")
        base = wid * CH
        cp1 = pltpu.async_copy(ea_hbm.at[pl.ds(base, CH)], vals_v, sem)
        cp2 = pltpu.async_copy(batch_hbm.at[pl.ds(base, CH)], ids_v, sem)

        zero16 = jnp.zeros((16,), jnp.float32)

        @pl.loop(0, B, step=16)
        def _(k):
            acc_v[pl.ds(k, 16)] = zero16

        cp1.wait()
        cp2.wait()

        iota = lax.iota(jnp.int32, 16)
        m_last = iota == 15
        m_first = iota == 0
        sh_next = jnp.minimum(iota + 1, 15)
        sh_prev = jnp.maximum(iota - 1, 0)

        @pl.loop(0, CH, step=16)
        def _(c):
            ids = ids_v[pl.ds(c, 16)]
            vals = vals_v[pl.ds(c, 16)]
            ids_n = plsc.load_gather(ids_v, [c + sh_next])
            ids_p = plsc.load_gather(ids_v, [c + sh_prev])
            bnd = (ids != ids_n) | m_last
            stt = (ids != ids_p) | m_first
            ps = plsc.cumsum(vals)
            plsc.addupdate_scatter(acc_v, [ids], ps, mask=bnd)
            plsc.addupdate_scatter(acc_v, [ids], vals - ps, mask=stt)

        pltpu.sync_copy(acc_v, out_hbm.at[pl.ds(wid * B, B)])

    return _seg_sum


@functools.lru_cache(maxsize=None)
def _build_ec_gather():
    @functools.partial(
        pl.kernel,
        mesh=plsc.VectorSubcoreMesh(**_SC_MESH),
        compiler_params=_sc_compiler_params(),
        out_type=(
            jax.ShapeDtypeStruct((B,), jnp.float32),
            jax.ShapeDtypeStruct((48,), jnp.float32),
            jax.ShapeDtypeStruct((N,), jnp.float32),
        ),
        scratch_types=[
            pltpu.VMEM((NW * B,), jnp.float32),
            pltpu.VMEM((NW * B,), jnp.float32),
            pltpu.VMEM((NW * 48,), jnp.float32),
            pltpu.VMEM((B,), jnp.float32),
            pltpu.VMEM((B,), jnp.float32),
            pltpu.VMEM((48,), jnp.float32),
            pltpu.VMEM((CH,), jnp.int32),
            pltpu.VMEM((CH,), jnp.float32),
            pltpu.SemaphoreType.DMA,
        ],
    )
    def _ec_gather(pk_hbm, pp_hbm, ppart_hbm, e0_hbm, batch_hbm,
                   ec_hbm, pout_hbm, eb_hbm,
                   pk_v, pp_v, ppart_v, e0_v, ec_v, pst_v, ids_v, eb_v, sem):
        wid = lax.axis_index("c") * 16 + lax.axis_index("s")
        base = wid * CH

        cps = [pltpu.async_copy(batch_hbm.at[pl.ds(base, CH)], ids_v, sem),
               pltpu.async_copy(pk_hbm, pk_v, sem),
               pltpu.async_copy(pp_hbm, pp_v, sem),
               pltpu.async_copy(ppart_hbm, ppart_v, sem),
               pltpu.async_copy(e0_hbm, e0_v, sem)]
        for cp in cps:
            cp.wait()

        zero16 = jnp.zeros((16,), jnp.float32)
        px = zero16
        py = zero16
        pz = zero16
        for w in range(NW):
            px = px + ppart_v[pl.ds(w * 48, 16)]
            py = py + ppart_v[pl.ds(w * 48 + 16, 16)]
            pz = pz + ppart_v[pl.ds(w * 48 + 32, 16)]
        px = zero16 + jnp.sum(px)
        py = zero16 + jnp.sum(py)
        pz = zero16 + jnp.sum(pz)

        @pl.loop(0, B, step=16)
        def _(k):
            a16 = -e0_v[pl.ds(k, 16)]
            for r in range(NW):
                a16 = a16 + pk_v[pl.ds(k + r * B, 16)]
                a16 = a16 + pp_v[pl.ds(k + r * B, 16)]
            ec_v[pl.ds(k, 16)] = a16

        pltpu.sync_copy(ec_v.at[pl.ds(wid * 32, 32)],
                        ec_hbm.at[pl.ds(wid * 32, 32)])

        @pl.when(wid == 0)
        def _():
            pst_v[pl.ds(0, 16)] = px
            pst_v[pl.ds(16, 16)] = py
            pst_v[pl.ds(32, 16)] = pz
            pltpu.sync_copy(pst_v, pout_hbm)

        @pl.loop(0, CH, step=16)
        def _(c):
            ids = ids_v[pl.ds(c, 16)]
            eb_v[pl.ds(c, 16)] = plsc.load_gather(ec_v, [ids])

        pltpu.sync_copy(eb_v, eb_hbm.at[pl.ds(base, CH)])

    return _ec_gather


def kernel(r, v, batch, z, m, emb, W1, b1, W2, b2, E0):
    f32 = jnp.float32
    bf16 = jnp.bfloat16
    r = r.astype(f32)
    v = v.astype(f32)
    rx3 = r[:, 0].reshape(NB, 1, A)
    ry3 = r[:, 1].reshape(NB, 1, A)
    rz3 = r[:, 2].reshape(NB, 1, A)
    vx = v[:, 0]
    vy = v[:, 1]
    vz = v[:, 2]
    batch_i = batch.astype(jnp.int32)
    z3 = z.astype(jnp.int32).reshape(NB, 1, A)
    m_f = m.astype(f32)

    embT = jnp.pad(emb.astype(f32).T, ((0, 0), (0, 128 - NZ)))
    W1eT = W1[3:].astype(f32).T
    W1rT = W1[:3].astype(f32).T
    w2c = W2.astype(f32)
    U = jnp.zeros((8, 128), f32)
    U = U.at[0, :64].set(W2.astype(f32)[:, 0])
    U = U.at[1:4, 64:].set(W1[:3].astype(f32))
    b1c = b1.astype(f32)[:, None]
    b2r = b2.astype(f32).reshape(1, 1)

    parts_kin, pparts = _build_kin_seg_sum()(batch_i, vx, vy, vz, m_f)

    ea3, gx3, gy3, gz3 = _tc_stage(rx3, ry3, rz3, z3, embT, W1eT, W1rT, U,
                                   w2c, b1c, b2r)
    parts_pot = _build_seg_sum()(ea3.reshape(N), batch_i)

    Ec, p48, Eb = _build_ec_gather()(parts_kin, parts_pot, pparts,
                                     E0.astype(f32), batch_i)

    JT = _tc2_stage(Eb.reshape(NB2, 1, A2), m_f.reshape(NB2, 1, A2),
                    gx3.reshape(NB2, 1, A2), gy3.reshape(NB2, 1, A2),
                    gz3.reshape(NB2, 1, A2), p48.reshape(1, 1, 48))

    p_vec = jnp.stack([p48[0], p48[16], p48[32]])
    c = jnp.concatenate([Ec, p_vec])
    return (c, JT.T)


# speedup vs baseline: 4.7034x; 1.0012x over previous
import dataclasses
import functools

import jax
import jax.numpy as jnp
from jax import lax
from jax.experimental import pallas as pl
from jax.experimental.pallas import tpu as pltpu
from jax.experimental.pallas import tpu_sc as plsc

N = 262144
B = 1024
NZ = 100
DE = 16
H = 64

A = 16384
NB = N // A
A2 = 8192
NB2 = N // A2
NW = 32
CH = N // NW

_SC_MESH = dict(core_axis_name="c", subcore_axis_name="s")


def _sc_compiler_params():
    cp = pltpu.CompilerParams()
    if "needs_layout_passes" in pltpu.CompilerParams.__dataclass_fields__:
        cp = dataclasses.replace(cp, needs_layout_passes=False)
    return cp


AH = A // 2


def _tc_body(rx_ref, ry_ref, rz_ref, z_ref, embT_ref, W1eT_ref, W1rT_ref,
             U_ref, w2c_ref, b1c_ref, b2_ref,
             ea_ref, gx_ref, gy_ref, gz_ref, MT_ref):
    i = pl.program_id(0)

    @pl.when(i == 0)
    def _():
        MT_ref[...] = lax.dot_general(W1eT_ref[...], embT_ref[...],
                                      (((1,), (0,)), ((), ())),
                                      preferred_element_type=jnp.float32)

    MT = MT_ref[...]
    w2c = w2c_ref[...]
    b1c = b1c_ref[...]

    for s in range(2):
        sl = slice(s * AH, (s + 1) * AH)
        rT = jnp.concatenate([rx_ref[0][:, sl], ry_ref[0][:, sl],
                              rz_ref[0][:, sl]], axis=0)
        z = z_ref[0][:, sl]

        idx = jnp.broadcast_to(z, (H, AH))
        acc = jnp.take_along_axis(MT, idx, axis=1)
        acc = acc + lax.dot_general(W1rT_ref[...], rT,
                                    (((1,), (0,)), ((), ())),
                                    preferred_element_type=jnp.float32)
        hpre = acc + b1c
        h = jnp.tanh(hpre)

        hw = h * w2c
        d = w2c - h * hw

        X = jnp.concatenate([h, d], axis=0)
        Y = lax.dot_general(U_ref[...], X, (((1,), (0,)), ((), ())),
                            preferred_element_type=jnp.float32)

        ea_ref[0, :, sl] = Y[0:1] + b2_ref[...]
        gx_ref[0, :, sl] = Y[1:2]
        gy_ref[0, :, sl] = Y[2:3]
        gz_ref[0, :, sl] = Y[3:4]


def _tc_stage(rx3, ry3, rz3, z3, embT, W1eT, W1rT, U, w2c, b1c, b2r):
    row = pl.BlockSpec((1, 1, A), lambda i: (i, 0, 0))
    full = lambda s: pl.BlockSpec(s, lambda i: tuple(0 for _ in s))
    out1 = jax.ShapeDtypeStruct((NB, 1, A), jnp.float32)
    return pl.pallas_call(
        _tc_body,
        grid=(NB,),
        in_specs=[row, row, row, row,
                  full((16, 128)), full((64, 16)), full((64, 3)),
                  full((8, 128)), full((64, 1)), full((64, 1)),
                  full((1, 1))],
        out_specs=[row, row, row, row],
        out_shape=[out1, out1, out1, out1],
        scratch_shapes=[pltpu.VMEM((64, 128), jnp.float32)],
    )(rx3, ry3, rz3, z3, embT, W1eT, W1rT, U, w2c, b1c, b2r)


def _tc2_body(eb_ref, m_ref, gx_ref, gy_ref, gz_ref, p_ref, jt_ref):
    eb = eb_ref[0]
    m2 = m_ref[0]
    a = eb + m2
    px = jnp.broadcast_to(p_ref[0, :, 0:1], (1, A2))
    py = jnp.broadcast_to(p_ref[0, :, 16:17], (1, A2))
    pz = jnp.broadcast_to(p_ref[0, :, 32:33], (1, A2))
    jt_ref[...] = jnp.concatenate(
        [gx_ref[0] * eb, gy_ref[0] * eb, gz_ref[0] * eb,
         a * px, a * py, a * pz], axis=0)


def _tc2_stage(eb3, m3, gx3, gy3, gz3, p48):
    row = pl.BlockSpec((1, 1, A2), lambda i: (i, 0, 0))
    return pl.pallas_call(
        _tc2_body,
        grid=(NB2,),
        in_specs=[row, row, row, row, row,
                  pl.BlockSpec((1, 1, 48), lambda i: (0, 0, 0))],
        out_specs=pl.BlockSpec((6, A2), lambda i: (0, i)),
        out_shape=jax.ShapeDtypeStruct((6, N), jnp.float32),
    )(eb3, m3, gx3, gy3, gz3, p48)


@functools.lru_cache(maxsize=None)
def _build_kin_seg_sum():
    @functools.partial(
        pl.kernel,
        mesh=plsc.VectorSubcoreMesh(**_SC_MESH),
        compiler_params=_sc_compiler_params(),
        out_type=(
            jax.ShapeDtypeStruct((NW * B,), jnp.float32),
            jax.ShapeDtypeStruct((NW * 48,), jnp.float32),
        ),
        scratch_types=[
            pltpu.VMEM((CH,), jnp.int32),
            pltpu.VMEM((CH,), jnp.float32),
            pltpu.VMEM((CH,), jnp.float32),
            pltpu.VMEM((CH,), jnp.float32),
            pltpu.VMEM((CH,), jnp.float32),
            pltpu.VMEM((B,), jnp.float32),
            pltpu.VMEM((48,), jnp.float32),
            pltpu.SemaphoreType.DMA,
        ],
    )
    def _kin_seg(batch_hbm, vx_hbm, vy_hbm, vz_hbm, m_hbm, out_hbm, pp_hbm,
                 ids_v, vx_v, vy_v, vz_v, m_v, acc_v, pst_v, sem):
        wid = lax.axis_index("c") * 16 + lax.axis_index("s")
        base = wid * CH
        cps = [pltpu.async_copy(batch_hbm.at[pl.ds(base, CH)], ids_v, sem),
               pltpu.async_copy(vx_hbm.at[pl.ds(base, CH)], vx_v, sem),
               pltpu.async_copy(vy_hbm.at[pl.ds(base, CH)], vy_v, sem),
               pltpu.async_copy(vz_hbm.at[pl.ds(base, CH)], vz_v, sem),
               pltpu.async_copy(m_hbm.at[pl.ds(base, CH)], m_v, sem)]

        zero16 = jnp.zeros((16,), jnp.float32)

        @pl.loop(0, B, step=16)
        def _(k):
            acc_v[pl.ds(k, 16)] = zero16

        pst_v[pl.ds(0, 16)] = zero16
        pst_v[pl.ds(16, 16)] = zero16
        pst_v[pl.ds(32, 16)] = zero16

        for cp in cps:
            cp.wait()

        iota = lax.iota(jnp.int32, 16)
        m_last = iota == 15
        m_first = iota == 0
        sh_next = jnp.minimum(iota + 1, 15)
        sh_prev = jnp.maximum(iota - 1, 0)

        def rb(x):
            b = lax.bitcast_convert_type(x, jnp.int32)
            r = (b + ((b >> 16) & 1) + jnp.int32(0x7FFF)) & jnp.int32(-65536)
            return lax.bitcast_convert_type(r, jnp.float32)

        @pl.loop(0, CH, step=16)
        def _(c):
            ids = ids_v[pl.ds(c, 16)]
            va = vx_v[pl.ds(c, 16)]
            vb = vy_v[pl.ds(c, 16)]
            vc = vz_v[pl.ds(c, 16)]
            mm = m_v[pl.ds(c, 16)]
            vals = 0.5 * mm * (va * va + vb * vb + vc * vc)
            ids_n = plsc.load_gather(ids_v, [c + sh_next])
            ids_p = plsc.load_gather(ids_v, [c + sh_prev])
            bnd = (ids != ids_n) | m_last
            stt = (ids != ids_p) | m_first
            ps = plsc.cumsum(vals)
            plsc.addupdate_scatter(acc_v, [ids], ps, mask=bnd)
            plsc.addupdate_scatter(acc_v, [ids], vals - ps, mask=stt)
            mb = rb(mm)
            pst_v[pl.ds(0, 16)] = pst_v[pl.ds(0, 16)] + mb * rb(va)
            pst_v[pl.ds(16, 16)] = pst_v[pl.ds(16, 16)] + mb * rb(vb)
            pst_v[pl.ds(32, 16)] = pst_v[pl.ds(32, 16)] + mb * rb(vc)

        pltpu.sync_copy(acc_v, out_hbm.at[pl.ds(wid * B, B)])
        pltpu.sync_copy(pst_v, pp_hbm.at[pl.ds(wid * 48, 48)])

    return _kin_seg


@functools.lru_cache(maxsize=None)
def _build_seg_sum():
    @functools.partial(
        pl.kernel,
        mesh=plsc.VectorSubcoreMesh(**_SC_MESH),
        compiler_params=_sc_compiler_params(),
        out_type=jax.ShapeDtypeStruct((NW * B,), jnp.float32),
        scratch_types=[
            pltpu.VMEM((CH,), jnp.float32),
            pltpu.VMEM((CH,), jnp.int32),
            pltpu.VMEM((B,), jnp.float32),
            pltpu.SemaphoreType.DMA,
        ],
    )
    def _seg_sum(ea_hbm, batch_hbm, out_hbm, vals_v, ids_v, acc_v, sem):
        wid = lax.axis_index("c") * 16 + lax.axis_index("s")
        base = wid * CH
        cp1 = pltpu.async_copy(ea_hbm.at[pl.ds(base, CH)], vals_v, sem)
        cp2 = pltpu.async_copy(batch_hbm.at[pl.ds(base, CH)], ids_v, sem)

        zero16 = jnp.zeros((16,), jnp.float32)

        @pl.loop(0, B, step=16)
        def _(k):
            acc_v[pl.ds(k, 16)] = zero16

        cp1.wait()
        cp2.wait()

        iota = lax.iota(jnp.int32, 16)
        m_last = iota == 15
        m_first = iota == 0
        sh_next = jnp.minimum(iota + 1, 15)
        sh_prev = jnp.maximum(iota - 1, 0)

        @pl.loop(0, CH, step=16)
        def _(c):
            ids = ids_v[pl.ds(c, 16)]
            vals = vals_v[pl.ds(c, 16)]
            ids_n = plsc.load_gather(ids_v, [c + sh_next])
            ids_p = plsc.load_gather(ids_v, [c + sh_prev])
            bnd = (ids != ids_n) | m_last
            stt = (ids != ids_p) | m_first
            ps = plsc.cumsum(vals)
            plsc.addupdate_scatter(acc_v, [ids], ps, mask=bnd)
            plsc.addupdate_scatter(acc_v, [ids], vals - ps, mask=stt)

        pltpu.sync_copy(acc_v, out_hbm.at[pl.ds(wid * B, B)])

    return _seg_sum


@functools.lru_cache(maxsize=None)
def _build_ec_gather():
    @functools.partial(
        pl.kernel,
        mesh=plsc.VectorSubcoreMesh(**_SC_MESH),
        compiler_params=_sc_compiler_params(),
        out_type=(
            jax.ShapeDtypeStruct((B,), jnp.float32),
            jax.ShapeDtypeStruct((48,), jnp.float32),
            jax.ShapeDtypeStruct((N,), jnp.float32),
        ),
        scratch_types=[
            pltpu.VMEM((NW * B,), jnp.float32),
            pltpu.VMEM((NW * B,), jnp.float32),
            pltpu.VMEM((NW * 48,), jnp.float32),
            pltpu.VMEM((B,), jnp.float32),
            pltpu.VMEM((B,), jnp.float32),
            pltpu.VMEM((48,), jnp.float32),
            pltpu.VMEM((CH,), jnp.int32),
            pltpu.VMEM((CH,), jnp.float32),
            pltpu.SemaphoreType.DMA,
        ],
    )
    def _ec_gather(pk_hbm, pp_hbm, ppart_hbm, e0_hbm, batch_hbm,
                   ec_hbm, pout_hbm, eb_hbm,
                   pk_v, pp_v, ppart_v, e0_v, ec_v, pst_v, ids_v, eb_v, sem):
        wid = lax.axis_index("c") * 16 + lax.axis_index("s")
        base = wid * CH

        cps = [pltpu.async_copy(batch_hbm.at[pl.ds(base, CH)], ids_v, sem),
               pltpu.async_copy(pk_hbm, pk_v, sem),
               pltpu.async_copy(pp_hbm, pp_v, sem),
               pltpu.async_copy(ppart_hbm, ppart_v, sem),
               pltpu.async_copy(e0_hbm, e0_v, sem)]
        for cp in cps:
            cp.wait()

        zero16 = jnp.zeros((16,), jnp.float32)
        px = zero16
        py = zero16
        pz = zero16
        for w in range(NW):
            px = px + ppart_v[pl.ds(w * 48, 16)]
            py = py + ppart_v[pl.ds(w * 48 + 16, 16)]
            pz = pz + ppart_v[pl.ds(w * 48 + 32, 16)]
        px = zero16 + jnp.sum(px)
        py = zero16 + jnp.sum(py)
        pz = zero16 + jnp.sum(pz)

        @pl.loop(0, B, step=16)
        def _(k):
            a16 = -e0_v[pl.ds(k, 16)]
            for r in range(NW):
                a16 = a16 + pk_v[pl.ds(k + r * B, 16)]
                a16 = a16 + pp_v[pl.ds(k + r * B, 16)]
            ec_v[pl.ds(k, 16)] = a16

        pltpu.sync_copy(ec_v.at[pl.ds(wid * 32, 32)],
                        ec_hbm.at[pl.ds(wid * 32, 32)])

        @pl.when(wid == 0)
        def _():
            pst_v[pl.ds(0, 16)] = px
            pst_v[pl.ds(16, 16)] = py
            pst_v[pl.ds(32, 16)] = pz
            pltpu.sync_copy(pst_v, pout_hbm)

        @pl.loop(0, CH, step=16)
        def _(c):
            ids = ids_v[pl.ds(c, 16)]
            eb_v[pl.ds(c, 16)] = plsc.load_gather(ec_v, [ids])

        pltpu.sync_copy(eb_v, eb_hbm.at[pl.ds(base, CH)])

    return _ec_gather


def kernel(r, v, batch, z, m, emb, W1, b1, W2, b2, E0):
    f32 = jnp.float32
    bf16 = jnp.bfloat16
    r = r.astype(f32)
    v = v.astype(f32)
    rx3 = r[:, 0].reshape(NB, 1, A)
    ry3 = r[:, 1].reshape(NB, 1, A)
    rz3 = r[:, 2].reshape(NB, 1, A)
    vx = v[:, 0]
    vy = v[:, 1]
    vz = v[:, 2]
    batch_i = batch.astype(jnp.int32)
    z3 = z.astype(jnp.int32).reshape(NB, 1, A)
    m_f = m.astype(f32)

    embT = jnp.pad(emb.astype(f32).T, ((0, 0), (0, 128 - NZ)))
    W1eT = W1[3:].astype(f32).T
    W1rT = W1[:3].astype(f32).T
    w2c = W2.astype(f32)
    U = jnp.zeros((8, 128), f32)
    U = U.at[0, :64].set(W2.astype(f32)[:, 0])
    U = U.at[1:4, 64:].set(W1[:3].astype(f32))
    b1c = b1.astype(f32)[:, None]
    b2r = b2.astype(f32).reshape(1, 1)

    parts_kin, pparts = _build_kin_seg_sum()(batch_i, vx, vy, vz, m_f)

    ea3, gx3, gy3, gz3 = _tc_stage(rx3, ry3, rz3, z3, embT, W1eT, W1rT, U,
                                   w2c, b1c, b2r)
    parts_pot = _build_seg_sum()(ea3.reshape(N), batch_i)

    Ec, p48, Eb = _build_ec_gather()(parts_kin, parts_pot, pparts,
                                     E0.astype(f32), batch_i)

    JT = _tc2_stage(Eb.reshape(NB2, 1, A2), m_f.reshape(NB2, 1, A2),
                    gx3.reshape(NB2, 1, A2), gy3.reshape(NB2, 1, A2),
                    gz3.reshape(NB2, 1, A2), p48.reshape(1, 1, 48))

    p_vec = jnp.stack([p48[0], p48[16], p48[32]])
    c = jnp.concatenate([Ec, p_vec])
    return (c, JT.T)
